# baseline (speedup 1.0000x reference)
import functools

import jax
import jax.numpy as jnp
from jax import lax
from jax.experimental import pallas as pl
from jax.experimental.pallas import tpu as pltpu
from jax.experimental.pallas import tpu_sc as plsc

BATCH = 16384
INPUT_DIM = 39
N_DENSE = 13
N_CAT = 26
EMB_DIM = 16
OUT_DIM = N_DENSE + N_CAT * EMB_DIM
GATHER_DIM = N_CAT * EMB_DIM
SLAB_OFF = 8
SLAB_W = INPUT_DIM - SLAB_OFF

NUM_CORES = 2
NUM_SUBCORES = 16
LANES = 16
NW = NUM_CORES * NUM_SUBCORES
BPW = BATCH // NW


def _gather_body(x32, *refs):
    tables = refs[:N_CAT]
    gout = refs[N_CAT]
    slab, idx2, gbuf, sem = refs[N_CAT + 1:]

    wid = lax.axis_index("s") * NUM_CORES + lax.axis_index("c")
    base = pl.multiple_of(wid * jnp.int32(BPW), BPW)

    pltpu.sync_copy(x32.at[pl.ds(base, BPW), pl.ds(SLAB_OFF, SLAB_W)], slab)

    lane = lax.iota(jnp.int32, LANES)

    def grp(c, carry):
        rows = lane + c * jnp.int32(LANES)
        c16 = pl.multiple_of(c * jnp.int32(LANES), LANES)
        for i in range(N_CAT):
            cols = jnp.full((LANES,), N_DENSE - SLAB_OFF + i, jnp.int32)
            idx2[jnp.int32(i), pl.ds(c16, LANES)] = plsc.load_gather(
                slab, [rows, cols]
            )
        return carry

    lax.fori_loop(0, BPW // LANES, grp, jnp.int32(0))

    for i in range(N_CAT):
        pltpu.async_copy(tables[i].at[idx2.at[jnp.int32(i)]], gbuf, sem).wait()
        pltpu.sync_copy(
            gbuf, gout.at[pl.ds(base, BPW), pl.ds(i * EMB_DIM, EMB_DIM)]
        )


def _assemble_body(x_ref, g_ref, o_ref):
    dense = x_ref[:, :N_DENSE].astype(jnp.float32)
    o_ref[...] = jnp.concatenate([dense, g_ref[...]], axis=1)


TC_ROWS = 512


@jax.jit
def _encode(x32, *tables):
    mesh = plsc.VectorSubcoreMesh(core_axis_name="c", subcore_axis_name="s")
    gout = pl.kernel(
        _gather_body,
        mesh=mesh,
        out_type=jax.ShapeDtypeStruct((BATCH, GATHER_DIM), jnp.float32),
        scratch_types=[
            pltpu.VMEM((BPW, SLAB_W), jnp.int32),
            pltpu.VMEM((N_CAT, BPW), jnp.int32),
            pltpu.VMEM((BPW, EMB_DIM), jnp.float32),
            pltpu.SemaphoreType.DMA,
        ],
        compiler_params=pltpu.CompilerParams(
            use_tc_tiling_on_sc=False, needs_layout_passes=False
        ),
    )(x32, *tables)

    return pl.pallas_call(
        _assemble_body,
        grid=(BATCH // TC_ROWS,),
        in_specs=[
            pl.BlockSpec((TC_ROWS, INPUT_DIM), lambda i: (i, 0)),
            pl.BlockSpec((TC_ROWS, GATHER_DIM), lambda i: (i, 0)),
        ],
        out_specs=pl.BlockSpec((TC_ROWS, OUT_DIM), lambda i: (i, 0)),
        out_shape=jax.ShapeDtypeStruct((BATCH, OUT_DIM), jnp.float32),
    )(x32, gout)


def kernel(x, emb_0, emb_1, emb_2, emb_3, emb_4, emb_5, emb_6, emb_7, emb_8,
           emb_9, emb_10, emb_11, emb_12, emb_13, emb_14, emb_15, emb_16,
           emb_17, emb_18, emb_19, emb_20, emb_21, emb_22, emb_23, emb_24,
           emb_25):
    with jax.enable_x64(False):
        x32 = jnp.asarray(x, jnp.int32)
        return _encode(x32, emb_0, emb_1, emb_2, emb_3, emb_4, emb_5, emb_6,
                       emb_7, emb_8, emb_9, emb_10, emb_11, emb_12, emb_13,
                       emb_14, emb_15, emb_16, emb_17, emb_18, emb_19, emb_20,
                       emb_21, emb_22, emb_23, emb_24, emb_25)


# speedup vs baseline: 1.0642x; 1.0642x over previous
import jax
import jax.numpy as jnp
from jax import lax
from jax.experimental import pallas as pl
from jax.experimental.pallas import tpu as pltpu
from jax.experimental.pallas import tpu_sc as plsc

BATCH = 16384
INPUT_DIM = 39
N_DENSE = 13
N_CAT = 26
EMB_DIM = 16
OUT_DIM = N_DENSE + N_CAT * EMB_DIM

NUM_CORES = 2
NUM_SUBCORES = 16
LANES = 16
NW = NUM_CORES * NUM_SUBCORES
BPW = BATCH // NW


def _encoder_body(xT, *refs):
    tables = refs[:N_CAT]
    outT = refs[N_CAT]
    idx2, dslab, dbuf, gbuf, tbuf, sem = refs[N_CAT + 1:]

    wid = lax.axis_index("s") * NUM_CORES + lax.axis_index("c")
    base = pl.multiple_of(wid * jnp.int32(BPW), BPW)
    lane = lax.iota(jnp.int32, LANES)

    pltpu.sync_copy(xT.at[pl.ds(N_DENSE, N_CAT), pl.ds(base, BPW)], idx2)
    pltpu.sync_copy(xT.at[pl.ds(0, N_DENSE), pl.ds(base, BPW)], dslab)

    def dgrp(c, carry):
        c16 = pl.multiple_of(c * jnp.int32(LANES), LANES)
        for j in range(N_DENSE):
            dbuf[jnp.int32(j), pl.ds(c16, LANES)] = dslab[
                jnp.int32(j), pl.ds(c16, LANES)
            ].astype(jnp.float32)
        return carry

    lax.fori_loop(0, BPW // LANES, dgrp, jnp.int32(0))
    pltpu.sync_copy(dbuf, outT.at[pl.ds(0, N_DENSE), pl.ds(base, BPW)])

    for i in range(N_CAT):
        pltpu.async_copy(tables[i].at[idx2.at[jnp.int32(i)]], gbuf, sem).wait()

        def tgrp(c, carry):
            c16 = pl.multiple_of(c * jnp.int32(LANES), LANES)
            rows = lane + c * jnp.int32(LANES)
            for e in range(EMB_DIM):
                cols = jnp.full((LANES,), e, jnp.int32)
                tbuf[jnp.int32(e), pl.ds(c16, LANES)] = plsc.load_gather(
                    gbuf, [rows, cols]
                )
            return carry

        lax.fori_loop(0, BPW // LANES, tgrp, jnp.int32(0))
        pltpu.sync_copy(
            tbuf, outT.at[pl.ds(N_DENSE + i * EMB_DIM, EMB_DIM), pl.ds(base, BPW)]
        )


@jax.jit
def _encode(xT, *tables):
    mesh = plsc.VectorSubcoreMesh(core_axis_name="c", subcore_axis_name="s")
    return pl.kernel(
        _encoder_body,
        mesh=mesh,
        out_type=jax.ShapeDtypeStruct((OUT_DIM, BATCH), jnp.float32),
        scratch_types=[
            pltpu.VMEM((N_CAT, BPW), jnp.int32),
            pltpu.VMEM((N_DENSE, BPW), jnp.int32),
            pltpu.VMEM((N_DENSE, BPW), jnp.float32),
            pltpu.VMEM((BPW, EMB_DIM), jnp.float32),
            pltpu.VMEM((EMB_DIM, BPW), jnp.float32),
            pltpu.SemaphoreType.DMA,
        ],
        compiler_params=pltpu.CompilerParams(
            use_tc_tiling_on_sc=False, needs_layout_passes=False
        ),
    )(xT, *tables)


def kernel(x, emb_0, emb_1, emb_2, emb_3, emb_4, emb_5, emb_6, emb_7, emb_8,
           emb_9, emb_10, emb_11, emb_12, emb_13, emb_14, emb_15, emb_16,
           emb_17, emb_18, emb_19, emb_20, emb_21, emb_22, emb_23, emb_24,
           emb_25):
    with jax.enable_x64(False):
        xT = jnp.asarray(x, jnp.int32).T
        outT = _encode(xT, emb_0, emb_1, emb_2, emb_3, emb_4, emb_5, emb_6,
                       emb_7, emb_8, emb_9, emb_10, emb_11, emb_12, emb_13,
                       emb_14, emb_15, emb_16, emb_17, emb_18, emb_19,
                       emb_20, emb_21, emb_22, emb_23, emb_24, emb_25)
        return outT.T
